# Initial kernel scaffold; baseline (speedup 1.0000x reference)
import jax
import jax.numpy as jnp
from jax.experimental import pallas as pl


def kernel(x, w1, b1, w2, b2):
    raise NotImplementedError("write your pallas kernel here")


# speedup vs baseline: 1.5058x; 1.5058x over previous
import functools

import jax
import jax.numpy as jnp
from jax import lax
from jax.experimental import pallas as pl
from jax.experimental.pallas import tpu as pltpu


def _fused_conv_conv_kernel(x_ref, w1_ref, b1_ref, w2_ref, b2_ref, o_ref, *,
                            H, W, Bt):
    P = H * W
    N = Bt * P

    pos = lax.broadcasted_iota(jnp.int32, (1, N), 1) % P
    row = pos // W
    col = pos % W
    offsets = [(ky - 1, kx - 1) for ky in range(3) for kx in range(3)]
    masks = []
    for oy, ox in offsets:
        if oy == 0 and ox == 0:
            masks.append(None)
        else:
            valid = ((row + oy >= 0) & (row + oy < H) &
                     (col + ox >= 0) & (col + ox < W))
            masks.append(valid.astype(jnp.bfloat16))

    def tap_slab(z):
        taps = []
        for t, (oy, ox) in enumerate(offsets):
            if oy == 0 and ox == 0:
                taps.append(z)
            else:
                shift = (-(oy * W + ox)) % N
                taps.append(pltpu.roll(z, shift=shift, axis=1) * masks[t])
        return jnp.concatenate(taps, axis=0)

    xb = jnp.concatenate(
        [x_ref[b].astype(jnp.bfloat16) for b in range(Bt)], axis=1)

    h = jnp.dot(w1_ref[...], tap_slab(xb),
                preferred_element_type=jnp.float32) + b1_ref[...]
    h = jnp.maximum(h, 0.0)

    y = jnp.dot(w2_ref[...], tap_slab(h.astype(jnp.bfloat16)),
                preferred_element_type=jnp.float32) + b2_ref[...]

    for b in range(Bt):
        o_ref[b] = y[:, b * P:(b + 1) * P].astype(o_ref.dtype)


def kernel(x, w1, b1, w2, b2):
    B, Cin, H, W = x.shape
    P = H * W
    C1 = w1.shape[-1]
    C2 = w2.shape[-1]
    Bt = 8 if B % 8 == 0 else 1

    w1m = jnp.transpose(w1, (3, 0, 1, 2)).reshape(C1, 9 * Cin)
    w2m = jnp.transpose(w2, (3, 0, 1, 2)).reshape(C2, 9 * C1)
    w1m = w1m.astype(jnp.bfloat16)
    w2m = w2m.astype(jnp.bfloat16)
    b1m = b1.reshape(C1, 1).astype(jnp.float32)
    b2m = b2.reshape(C2, 1).astype(jnp.float32)

    x_flat = x.reshape(B, Cin, P)

    kernel_fn = functools.partial(_fused_conv_conv_kernel, H=H, W=W, Bt=Bt)

    N = Bt * P
    io_bytes = 2 * (Bt * Cin * P * 4 + Bt * C2 * P * 4)
    slab_bytes = (9 * Cin + 9 * C1) * N * 2 + (Cin + C1 + C2) * N * 4
    vmem_limit = int(min(64 << 20, max(16 << 20, 2 * io_bytes + slab_bytes)))

    flops = 2 * B * P * 9 * (Cin * C1 + C1 * C2)
    bytes_accessed = int(B * (Cin + C2) * P * 4)

    out_flat = pl.pallas_call(
        kernel_fn,
        out_shape=jax.ShapeDtypeStruct((B, C2, P), x.dtype),
        grid_spec=pltpu.PrefetchScalarGridSpec(
            num_scalar_prefetch=0,
            grid=(B // Bt,),
            in_specs=[
                pl.BlockSpec((Bt, Cin, P), lambda g: (g, 0, 0)),
                pl.BlockSpec((C1, 9 * Cin), lambda g: (0, 0)),
                pl.BlockSpec((C1, 1), lambda g: (0, 0)),
                pl.BlockSpec((C2, 9 * C1), lambda g: (0, 0)),
                pl.BlockSpec((C2, 1), lambda g: (0, 0)),
            ],
            out_specs=pl.BlockSpec((Bt, C2, P), lambda g: (g, 0, 0)),
        ),
        compiler_params=pltpu.CompilerParams(
            dimension_semantics=("parallel",),
            vmem_limit_bytes=vmem_limit),
        cost_estimate=pl.CostEstimate(
            flops=flops, transcendentals=0, bytes_accessed=bytes_accessed),
    )(x_flat, w1m, b1m, w2m, b2m)

    return out_flat.reshape(B, C2, H, W)


# speedup vs baseline: 1.7524x; 1.1638x over previous
import functools

import jax
import jax.numpy as jnp
from jax import lax
from jax.experimental import pallas as pl
from jax.experimental.pallas import tpu as pltpu


def _fused_conv_conv_kernel(x_ref, w1_ref, b1_ref, w2_ref, b2_ref, o_ref, *,
                            H, W, Bt, C1, C2):
    P = H * W
    N = Bt * P
    C2p = b2_ref.shape[0]

    pos = lax.broadcasted_iota(jnp.int32, (1, N), 1) % P
    row = pos // W
    col = pos % W
    cmask_l = (col >= 1).astype(jnp.bfloat16)
    cmask_r = (col <= W - 2).astype(jnp.bfloat16)
    rmask_u = (row >= 1).astype(jnp.float32)
    rmask_d = (row <= H - 2).astype(jnp.float32)

    def colslab(z):
        zl = pltpu.roll(z, shift=1, axis=1) * cmask_l
        zr = pltpu.roll(z, shift=N - 1, axis=1) * cmask_r
        return jnp.concatenate([zl, z, zr], axis=0)

    def row_combine(z, c):
        up = pltpu.roll(z[0:c], shift=W, axis=1) * rmask_u
        dn = pltpu.roll(z[2 * c:3 * c], shift=N - W, axis=1) * rmask_d
        return z[c:2 * c] + up + dn

    xb = jnp.concatenate(
        [x_ref[b].astype(jnp.bfloat16) for b in range(Bt)], axis=1)

    z1 = jnp.dot(w1_ref[...], colslab(xb),
                 preferred_element_type=jnp.float32)
    h = row_combine(z1, C1) + b1_ref[...]
    h = jnp.maximum(h, 0.0)

    z2 = jnp.dot(w2_ref[...], colslab(h.astype(jnp.bfloat16)),
                 preferred_element_type=jnp.float32)
    y = row_combine(z2, C2p) + b2_ref[...]

    for b in range(Bt):
        o_ref[b] = y[0:C2, b * P:(b + 1) * P].astype(o_ref.dtype)


def kernel(x, w1, b1, w2, b2):
    B, Cin, H, W = x.shape
    P = H * W
    C1 = w1.shape[-1]
    C2 = w2.shape[-1]
    C2p = (C2 + 7) // 8 * 8
    Bt = 8 if B % 8 == 0 else 1

    w1m = jnp.transpose(w1, (0, 3, 1, 2)).reshape(3 * C1, 3 * Cin)
    w2t = jnp.transpose(w2, (0, 3, 1, 2))
    w2t = jnp.pad(w2t, ((0, 0), (0, C2p - C2), (0, 0), (0, 0)))
    w2m = w2t.reshape(3 * C2p, 3 * C1)
    w1m = w1m.astype(jnp.bfloat16)
    w2m = w2m.astype(jnp.bfloat16)
    b1m = b1.reshape(C1, 1).astype(jnp.float32)
    b2m = jnp.pad(b2, (0, C2p - C2)).reshape(C2p, 1).astype(jnp.float32)

    x_flat = x.reshape(B, Cin, P)

    kernel_fn = functools.partial(_fused_conv_conv_kernel, H=H, W=W, Bt=Bt,
                                  C1=C1, C2=C2)

    N = Bt * P
    io_bytes = 2 * (Bt * Cin * P * 4 + Bt * C2 * P * 4)
    work_bytes = (3 * Cin + 3 * C1 + Cin + C1) * N * 2 \
        + (3 * C1 + 3 * C2p + C1 + C2p) * N * 4
    vmem_limit = int(min(64 << 20, max(16 << 20, 2 * io_bytes + work_bytes)))

    flops = 2 * B * P * 9 * (Cin * C1 + C1 * C2p)
    bytes_accessed = int(B * (Cin + C2) * P * 4)

    out_flat = pl.pallas_call(
        kernel_fn,
        out_shape=jax.ShapeDtypeStruct((B, C2, P), x.dtype),
        grid_spec=pltpu.PrefetchScalarGridSpec(
            num_scalar_prefetch=0,
            grid=(B // Bt,),
            in_specs=[
                pl.BlockSpec((Bt, Cin, P), lambda g: (g, 0, 0)),
                pl.BlockSpec((3 * C1, 3 * Cin), lambda g: (0, 0)),
                pl.BlockSpec((C1, 1), lambda g: (0, 0)),
                pl.BlockSpec((3 * C2p, 3 * C1), lambda g: (0, 0)),
                pl.BlockSpec((C2p, 1), lambda g: (0, 0)),
            ],
            out_specs=pl.BlockSpec((Bt, C2, P), lambda g: (g, 0, 0)),
        ),
        compiler_params=pltpu.CompilerParams(
            dimension_semantics=("parallel",),
            vmem_limit_bytes=vmem_limit),
        cost_estimate=pl.CostEstimate(
            flops=flops, transcendentals=0, bytes_accessed=bytes_accessed),
    )(x_flat, w1m, b1m, w2m, b2m)

    return out_flat.reshape(B, C2, H, W)


# speedup vs baseline: 1.9837x; 1.1320x over previous
import functools

import jax
import jax.numpy as jnp
from jax import lax
from jax.experimental import pallas as pl
from jax.experimental.pallas import tpu as pltpu


def _fused_conv_conv_kernel(x_ref, w1_ref, b1_ref, w2_ref, b2_ref, o_ref, *,
                            H, W, Bt, C1, C2):
    P = H * W
    N = Bt * P
    C2p = b2_ref.shape[0]

    pos = lax.broadcasted_iota(jnp.int32, (1, N), 1) % P
    row = pos // W
    col = pos % W
    cmask_l = (col >= 1).astype(jnp.bfloat16)
    cmask_r = (col <= W - 2).astype(jnp.bfloat16)
    rmask_u = (row >= 1).astype(jnp.bfloat16)
    rmask_d = (row <= H - 2).astype(jnp.bfloat16)

    def colslab(z):
        zl = pltpu.roll(z, shift=1, axis=1) * cmask_l
        zr = pltpu.roll(z, shift=N - 1, axis=1) * cmask_r
        return jnp.concatenate([zl, z, zr], axis=0)

    def row_combine(z, c, b):
        up = pltpu.roll(z[0:c].astype(jnp.bfloat16), shift=W,
                        axis=1) * rmask_u
        dn = pltpu.roll(z[2 * c:3 * c].astype(jnp.bfloat16), shift=N - W,
                        axis=1) * rmask_d
        return z[c:2 * c] + (up + dn).astype(jnp.float32) + b

    xb = jnp.concatenate(
        [x_ref[b].astype(jnp.bfloat16) for b in range(Bt)], axis=1)

    z1 = jnp.dot(w1_ref[...], colslab(xb),
                 preferred_element_type=jnp.float32)
    h = row_combine(z1, C1, b1_ref[...])
    h = jnp.maximum(h, 0.0)

    z2 = jnp.dot(w2_ref[...], colslab(h.astype(jnp.bfloat16)),
                 preferred_element_type=jnp.float32)
    y = row_combine(z2, C2p, b2_ref[...])

    for b in range(Bt):
        o_ref[b] = y[0:C2, b * P:(b + 1) * P].astype(o_ref.dtype)


def kernel(x, w1, b1, w2, b2):
    B, Cin, H, W = x.shape
    P = H * W
    C1 = w1.shape[-1]
    C2 = w2.shape[-1]
    C2p = (C2 + 7) // 8 * 8
    Bt = 8 if B % 8 == 0 else 1

    w1m = jnp.transpose(w1, (0, 3, 1, 2)).reshape(3 * C1, 3 * Cin)
    w2t = jnp.transpose(w2, (0, 3, 1, 2))
    w2t = jnp.pad(w2t, ((0, 0), (0, C2p - C2), (0, 0), (0, 0)))
    w2m = w2t.reshape(3 * C2p, 3 * C1)
    w1m = w1m.astype(jnp.bfloat16)
    w2m = w2m.astype(jnp.bfloat16)
    b1m = b1.reshape(C1, 1).astype(jnp.float32)
    b2m = jnp.pad(b2, (0, C2p - C2)).reshape(C2p, 1).astype(jnp.float32)

    x_flat = x.reshape(B, Cin, P)

    kernel_fn = functools.partial(_fused_conv_conv_kernel, H=H, W=W, Bt=Bt,
                                  C1=C1, C2=C2)

    N = Bt * P
    io_bytes = 2 * (Bt * Cin * P * 4 + Bt * C2 * P * 4)
    work_bytes = (3 * Cin + 3 * C1 + Cin + C1) * N * 2 \
        + (3 * C1 + 3 * C2p + C1 + C2p) * N * 4
    vmem_limit = int(min(64 << 20, max(16 << 20, 2 * io_bytes + work_bytes)))

    flops = 2 * B * P * 9 * (Cin * C1 + C1 * C2p)
    bytes_accessed = int(B * (Cin + C2) * P * 4)

    out_flat = pl.pallas_call(
        kernel_fn,
        out_shape=jax.ShapeDtypeStruct((B, C2, P), x.dtype),
        grid_spec=pltpu.PrefetchScalarGridSpec(
            num_scalar_prefetch=0,
            grid=(B // Bt,),
            in_specs=[
                pl.BlockSpec((Bt, Cin, P), lambda g: (g, 0, 0)),
                pl.BlockSpec((3 * C1, 3 * Cin), lambda g: (0, 0)),
                pl.BlockSpec((C1, 1), lambda g: (0, 0)),
                pl.BlockSpec((3 * C2p, 3 * C1), lambda g: (0, 0)),
                pl.BlockSpec((C2p, 1), lambda g: (0, 0)),
            ],
            out_specs=pl.BlockSpec((Bt, C2, P), lambda g: (g, 0, 0)),
        ),
        compiler_params=pltpu.CompilerParams(
            dimension_semantics=("parallel",),
            vmem_limit_bytes=vmem_limit),
        cost_estimate=pl.CostEstimate(
            flops=flops, transcendentals=0, bytes_accessed=bytes_accessed),
    )(x_flat, w1m, b1m, w2m, b2m)

    return out_flat.reshape(B, C2, H, W)


# speedup vs baseline: 2.0360x; 1.0264x over previous
import functools

import jax
import jax.numpy as jnp
from jax import lax
from jax.experimental import pallas as pl
from jax.experimental.pallas import tpu as pltpu


def _fused_conv_conv_kernel(x_ref, w1_ref, b1_ref, w2_ref, b2_ref, o_ref, *,
                            H, W, Bt, C1, C2):
    P = H * W
    N = Bt * P
    C2p = b2_ref.shape[0]

    pos = lax.broadcasted_iota(jnp.int32, (1, N), 1) % P
    row = pos // W
    col = pos % W
    cmask_l = (col >= 1).astype(jnp.bfloat16)
    cmask_r = (col <= W - 2).astype(jnp.bfloat16)
    rmask_u = (row >= 1).astype(jnp.bfloat16)
    rmask_d = (row <= H - 2).astype(jnp.bfloat16)

    def colslab(z):
        zl = pltpu.roll(z, shift=1, axis=1) * cmask_l
        zr = pltpu.roll(z, shift=N - 1, axis=1) * cmask_r
        return jnp.concatenate([zl, z, zr], axis=0)

    def conv(w_ref, s, c, b):
        zm = jnp.dot(w_ref[0:c], s, preferred_element_type=jnp.float32)
        zud = jnp.dot(w_ref[c:3 * c], s,
                      preferred_element_type=jnp.float32)
        zud = zud.astype(jnp.bfloat16)
        up = pltpu.roll(zud[0:c], shift=W, axis=1) * rmask_u
        dn = pltpu.roll(zud[c:2 * c], shift=N - W, axis=1) * rmask_d
        return zm + (up + dn).astype(jnp.float32) + b

    xb = jnp.concatenate(
        [x_ref[b].astype(jnp.bfloat16) for b in range(Bt)], axis=1)

    h = conv(w1_ref, colslab(xb), C1, b1_ref[...])
    h = jnp.maximum(h, 0.0)

    y = conv(w2_ref, colslab(h.astype(jnp.bfloat16)), C2p, b2_ref[...])

    for b in range(Bt):
        o_ref[b] = y[0:C2, b * P:(b + 1) * P].astype(o_ref.dtype)


def kernel(x, w1, b1, w2, b2):
    B, Cin, H, W = x.shape
    P = H * W
    C1 = w1.shape[-1]
    C2 = w2.shape[-1]
    C2p = (C2 + 7) // 8 * 8
    Bt = 8 if B % 8 == 0 else 1

    ky_order = jnp.array([1, 0, 2])
    w1m = jnp.transpose(w1, (0, 3, 1, 2))[ky_order].reshape(3 * C1, 3 * Cin)
    w2t = jnp.transpose(w2, (0, 3, 1, 2))[ky_order]
    w2t = jnp.pad(w2t, ((0, 0), (0, C2p - C2), (0, 0), (0, 0)))
    w2m = w2t.reshape(3 * C2p, 3 * C1)
    w1m = w1m.astype(jnp.bfloat16)
    w2m = w2m.astype(jnp.bfloat16)
    b1m = b1.reshape(C1, 1).astype(jnp.float32)
    b2m = jnp.pad(b2, (0, C2p - C2)).reshape(C2p, 1).astype(jnp.float32)

    x_flat = x.reshape(B, Cin, P)

    kernel_fn = functools.partial(_fused_conv_conv_kernel, H=H, W=W, Bt=Bt,
                                  C1=C1, C2=C2)

    N = Bt * P
    io_bytes = 2 * (Bt * Cin * P * 4 + Bt * C2 * P * 4)
    work_bytes = (3 * Cin + 3 * C1 + Cin + C1) * N * 2 \
        + (3 * C1 + 3 * C2p + C1 + C2p) * N * 4
    vmem_limit = int(min(64 << 20, max(16 << 20, 2 * io_bytes + work_bytes)))

    flops = 2 * B * P * 9 * (Cin * C1 + C1 * C2p)
    bytes_accessed = int(B * (Cin + C2) * P * 4)

    out_flat = pl.pallas_call(
        kernel_fn,
        out_shape=jax.ShapeDtypeStruct((B, C2, P), x.dtype),
        grid_spec=pltpu.PrefetchScalarGridSpec(
            num_scalar_prefetch=0,
            grid=(B // Bt,),
            in_specs=[
                pl.BlockSpec((Bt, Cin, P), lambda g: (g, 0, 0)),
                pl.BlockSpec((3 * C1, 3 * Cin), lambda g: (0, 0)),
                pl.BlockSpec((C1, 1), lambda g: (0, 0)),
                pl.BlockSpec((3 * C2p, 3 * C1), lambda g: (0, 0)),
                pl.BlockSpec((C2p, 1), lambda g: (0, 0)),
            ],
            out_specs=pl.BlockSpec((Bt, C2, P), lambda g: (g, 0, 0)),
        ),
        compiler_params=pltpu.CompilerParams(
            dimension_semantics=("parallel",),
            vmem_limit_bytes=vmem_limit),
        cost_estimate=pl.CostEstimate(
            flops=flops, transcendentals=0, bytes_accessed=bytes_accessed),
    )(x_flat, w1m, b1m, w2m, b2m)

    return out_flat.reshape(B, C2, H, W)
